# baseline (speedup 1.0000x reference)
import functools

import jax
import jax.numpy as jnp
from jax import lax
from jax.experimental import pallas as pl
from jax.experimental.pallas import tpu as pltpu
from jax.experimental.pallas import tpu_sc as plsc

_D = 64
_BT = 128
_G = 2
_L = 16
_WBS = _BT + 1


@functools.partial(jax.jit, static_argnums=(0, 1, 2))
def _lookup_call(hist, nbt, num_rows, j4, map_packed, table):
    info = plsc.get_sparse_core_info()
    nw = info.num_cores * info.num_subcores
    ngt = nbt // _G
    n_st = hist * ngt
    assert n_st % (2 * nw) == 0, (hist, nbt, nw)
    st_per_w = n_st // nw
    map_words = map_packed.shape[0]
    mesh = plsc.VectorSubcoreMesh(core_axis_name="c", subcore_axis_name="s")

    @functools.partial(
        pl.kernel,
        mesh=mesh,
        compiler_params=pltpu.CompilerParams(use_tc_tiling_on_sc=False,
                                             needs_layout_passes=False),
        out_type=jax.ShapeDtypeStruct((hist, _D // 8, nbt, 8, _BT),
                                      jnp.float32),
        scratch_types=[
            pltpu.VMEM((map_words,), jnp.int32),
            pltpu.VMEM((2, _G, 2, _BT), jnp.int32),
            pltpu.VMEM((2, _G, _BT), jnp.int32),
            pltpu.VMEM((2, _G * _BT, _D), jnp.float32),
            pltpu.VMEM((2, _D // 8, _G, 8, _WBS), jnp.float32),
            pltpu.SemaphoreType.DMA,
            pltpu.SemaphoreType.DMA,
            pltpu.SemaphoreType.DMA,
            pltpu.SemaphoreType.DMA,
            pltpu.SemaphoreType.DMA,
        ],
    )
    def lookup(j4_hbm, map_hbm, table_hbm, out_hbm,
               map_v, idx_v, row_v, rows_v, wb_v, sem_idx, sem_g0, sem_g1,
               sem_o0, sem_o1):
        wid = lax.axis_index("s") * info.num_cores + lax.axis_index("c")
        s0 = wid * st_per_w
        iota = lax.iota(jnp.int32, _L)
        dts = [lax.shift_right_logical(iota + d0, 3) for d0 in range(0, _D, _L)]
        dss = [jnp.bitwise_and(iota + d0, 7) for d0 in range(0, _D, _L)]
        sem_o = (sem_o0, sem_o1)
        sem_g = (sem_g0, sem_g1)

        pltpu.async_copy(j4_hbm.at[s0 // ngt, pl.ds((s0 % ngt) * _G, _G)],
                         idx_v.at[0], sem_idx)
        pltpu.sync_copy(map_hbm, map_v)

        def compute_rowids(k, b):
            pltpu.make_async_copy(
                j4_hbm.at[0, pl.ds(0, _G)], idx_v.at[b], sem_idx).wait()
            sn = s0 + jnp.minimum(k + 1, st_per_w - 1)
            pltpu.async_copy(
                j4_hbm.at[sn // ngt, pl.ds((sn % ngt) * _G, _G)],
                idx_v.at[1 - b], sem_idx)
            for g in range(_G):
                for t in range(_BT // _L):
                    sl = pl.ds(t * _L, _L)
                    v0 = idx_v[b, g, 0, sl]
                    v1 = idx_v[b, g, 1, sl]
                    lin = v0 * 8 + v1
                    word = plsc.load_gather(
                        map_v, [lax.shift_right_logical(lin, 1)])
                    hi = lax.shift_right_logical(word, 16)
                    lo = jnp.bitwise_and(word, 0xFFFF)
                    m = jnp.where(jnp.bitwise_and(lin, 1) == 1, hi, lo)
                    m = jnp.minimum(m, num_rows - 4)
                    spec = (num_rows - 3) + jnp.minimum(
                        jnp.maximum(v0, 0), 2)
                    row_v[b, g, sl] = jnp.where(v0 <= 2, spec, m)

        def fire_gather(b):
            for g in range(_G):
                pltpu.async_copy(table_hbm.at[row_v.at[b, g]],
                                 rows_v.at[b, pl.ds(g * _BT, _BT)], sem_g[b])

        def drain_gather(b):
            for g in range(_G):
                pltpu.make_async_copy(
                    table_hbm.at[pl.ds(0, _BT)],
                    rows_v.at[b, pl.ds(g * _BT, _BT)], sem_g[b]).wait()

        def drain_writeout(w):
            pltpu.make_async_copy(
                wb_v.at[w, :, :, :, pl.ds(0, _BT)],
                out_hbm.at[0, :, pl.ds(0, _G)], sem_o[w]).wait()

        def transpose_and_write(k, b):
            for g in range(_G):
                gcst = jnp.int32(g)

                def tbody(blk, carry):
                    for j in range(_L):
                        bl = blk * _L + j
                        col = iota * 0 + bl
                        gv = iota * 0 + gcst
                        for i, d0 in enumerate(range(0, _D, _L)):
                            v = rows_v[b, g * _BT + bl, pl.ds(d0, _L)]
                            plsc.store_scatter(
                                wb_v.at[b], [dts[i], gv, dss[i], col], v)
                    return carry
                lax.fori_loop(0, _BT // _L, tbody, 0)
            s = s0 + k
            h = s // ngt
            bt = (s % ngt) * _G
            pltpu.async_copy(wb_v.at[b, :, :, :, pl.ds(0, _BT)],
                             out_hbm.at[h, :, pl.ds(bt, _G)], sem_o[b])

        def stage(i, k, b, min_i1, min_i2):
            compute_rowids(k, b)
            fire_gather(b)

            @pl.when(i >= min_i1)
            def _():
                drain_gather(1 - b)

                @pl.when(i >= min_i2)
                def _():
                    drain_writeout(1 - b)
                transpose_and_write(k - 1, 1 - b)

        def pair_body(i, carry):
            stage(i, 2 * i, 0, 1, 2)
            stage(i, 2 * i + 1, 1, 0, 1)
            return carry

        lax.fori_loop(0, st_per_w // 2, pair_body, 0)
        drain_gather(1)
        drain_writeout(1)
        transpose_and_write(st_per_w - 1, 1)
        drain_writeout(0)
        drain_writeout(1)
        pltpu.make_async_copy(
            j4_hbm.at[0, pl.ds(0, _G)], idx_v.at[0], sem_idx).wait()

    return lookup(j4, map_packed, table)


def kernel(idx, special_table, attached_table, index_map):
    b, hist, _ = idx.shape
    nbt = b // _BT
    j4 = (idx.astype(jnp.int32)
          .transpose(1, 2, 0)
          .reshape(hist, 2, nbt, _BT)
          .transpose(0, 2, 1, 3))
    a = attached_table.shape[0]
    table = jnp.concatenate(
        [attached_table.astype(jnp.float32), special_table.astype(jnp.float32)],
        axis=0)
    mp = index_map.reshape(-1).astype(jnp.uint32)
    map_packed = (mp[0::2] | (mp[1::2] << 16)).astype(jnp.int32)
    u = _lookup_call(hist, nbt, a + 3, j4, map_packed, table)
    return u.transpose(2, 4, 0, 1, 3).reshape(b, hist, _D)


# speedup vs baseline: 1.0578x; 1.0578x over previous
import functools

import jax
import jax.numpy as jnp
from jax import lax
from jax.experimental import pallas as pl
from jax.experimental.pallas import tpu as pltpu
from jax.experimental.pallas import tpu_sc as plsc

_D = 64
_BT = 128
_L = 16
_WBS = _BT + 1


@functools.partial(jax.jit, static_argnums=(0, 1, 2))
def _lookup_call(hist, nbt, num_rows, j4, map_packed, table):
    info = plsc.get_sparse_core_info()
    nw = info.num_cores * info.num_subcores
    n_st = hist * nbt
    assert n_st % (2 * nw) == 0, (hist, nbt, nw)
    st_per_w = n_st // nw
    map_words = map_packed.shape[0]
    mesh = plsc.VectorSubcoreMesh(core_axis_name="c", subcore_axis_name="s")

    @functools.partial(
        pl.kernel,
        mesh=mesh,
        compiler_params=pltpu.CompilerParams(use_tc_tiling_on_sc=False,
                                             needs_layout_passes=False),
        out_type=jax.ShapeDtypeStruct((hist, _D // 8, nbt, 8, _BT),
                                      jnp.float32),
        scratch_types=[
            pltpu.VMEM((map_words,), jnp.int32),
            pltpu.VMEM((2, 2, _BT), jnp.int32),
            pltpu.VMEM((2, _BT), jnp.int32),
            pltpu.VMEM((2, _BT, _D), jnp.float32),
            pltpu.VMEM((2, _D // 8, 8, _WBS), jnp.float32),
            pltpu.SemaphoreType.DMA,
            pltpu.SemaphoreType.DMA,
            pltpu.SemaphoreType.DMA,
            pltpu.SemaphoreType.DMA,
            pltpu.SemaphoreType.DMA,
        ],
    )
    def lookup(j4_hbm, map_hbm, table_hbm, out_hbm,
               map_v, idx_v, row_v, rows_v, wb_v, sem_idx, sem_g0, sem_g1,
               sem_o0, sem_o1):
        wid = lax.axis_index("s") * info.num_cores + lax.axis_index("c")
        s0 = wid * st_per_w
        iota = lax.iota(jnp.int32, _L)
        dts = [lax.shift_right_logical(iota + d0, 3) for d0 in range(0, _D, _L)]
        dss = [jnp.bitwise_and(iota + d0, 7) for d0 in range(0, _D, _L)]
        sem_o = (sem_o0, sem_o1)
        sem_g = (sem_g0, sem_g1)

        pltpu.async_copy(j4_hbm.at[s0 // nbt, s0 % nbt], idx_v.at[0], sem_idx)
        pltpu.sync_copy(map_hbm, map_v)

        def compute_rowids(k, b):
            pltpu.make_async_copy(
                j4_hbm.at[0, 0], idx_v.at[b], sem_idx).wait()
            sn = s0 + jnp.minimum(k + 1, st_per_w - 1)
            pltpu.async_copy(j4_hbm.at[sn // nbt, sn % nbt], idx_v.at[1 - b],
                             sem_idx)
            for t in range(_BT // _L):
                sl = pl.ds(t * _L, _L)
                v0 = idx_v[b, 0, sl]
                v1 = idx_v[b, 1, sl]
                lin = v0 * 8 + v1
                word = plsc.load_gather(
                    map_v, [lax.shift_right_logical(lin, 1)])
                hi = lax.shift_right_logical(word, 16)
                lo = jnp.bitwise_and(word, 0xFFFF)
                m = jnp.where(jnp.bitwise_and(lin, 1) == 1, hi, lo)
                m = jnp.minimum(m, num_rows - 4)
                spec = (num_rows - 3) + jnp.minimum(jnp.maximum(v0, 0), 2)
                row_v[b, sl] = jnp.where(v0 <= 2, spec, m)

        def fire_gather(b):
            pltpu.async_copy(table_hbm.at[row_v.at[b]], rows_v.at[b], sem_g[b])

        def drain_gather(b):
            pltpu.make_async_copy(
                table_hbm.at[pl.ds(0, _BT)], rows_v.at[b], sem_g[b]).wait()

        def drain_writeout(w):
            pltpu.make_async_copy(
                wb_v.at[w, :, :, pl.ds(0, _BT)],
                out_hbm.at[0, :, 0], sem_o[w]).wait()

        def transpose_and_write(k, b):
            def tbody(blk, carry):
                for j in range(_L):
                    bl = blk * _L + j
                    col = iota * 0 + bl
                    for i, d0 in enumerate(range(0, _D, _L)):
                        v = rows_v[b, bl, pl.ds(d0, _L)]
                        plsc.store_scatter(wb_v.at[b], [dts[i], dss[i], col],
                                           v)
                return carry
            lax.fori_loop(0, _BT // _L, tbody, 0)
            s = s0 + k
            h = s // nbt
            bt = s % nbt
            pltpu.async_copy(wb_v.at[b, :, :, pl.ds(0, _BT)],
                             out_hbm.at[h, :, bt], sem_o[b])

        def stage(i, k, b, min_i1, min_i2):
            compute_rowids(k, b)
            fire_gather(b)

            @pl.when(i >= min_i1)
            def _():
                drain_gather(1 - b)

                @pl.when(i >= min_i2)
                def _():
                    drain_writeout(1 - b)
                transpose_and_write(k - 1, 1 - b)

        def pair_body(i, carry):
            stage(i, 2 * i, 0, 1, 2)
            stage(i, 2 * i + 1, 1, 0, 1)
            return carry

        lax.fori_loop(0, st_per_w // 2, pair_body, 0)
        drain_gather(1)
        drain_writeout(1)
        transpose_and_write(st_per_w - 1, 1)
        drain_writeout(0)
        drain_writeout(1)
        pltpu.make_async_copy(j4_hbm.at[0, 0], idx_v.at[0], sem_idx).wait()

    return lookup(j4, map_packed, table)


def kernel(idx, special_table, attached_table, index_map):
    b, hist, _ = idx.shape
    nbt = b // _BT
    j4 = (idx.astype(jnp.int32)
          .transpose(1, 2, 0)
          .reshape(hist, 2, nbt, _BT)
          .transpose(0, 2, 1, 3))
    a = attached_table.shape[0]
    table = jnp.concatenate(
        [attached_table.astype(jnp.float32), special_table.astype(jnp.float32)],
        axis=0)
    mp = index_map.reshape(-1).astype(jnp.uint32)
    map_packed = (mp[0::2] | (mp[1::2] << 16)).astype(jnp.int32)
    u = _lookup_call(hist, nbt, a + 3, j4, map_packed, table)
    return u.transpose(2, 4, 0, 1, 3).reshape(b, hist, _D)
